# baseline (speedup 1.0000x reference)
import functools

import jax
import jax.numpy as jnp
from jax import lax
from jax.experimental import pallas as pl
from jax.experimental.pallas import tpu as pltpu

_D = 2048
_T = 2048
_OUT = 3072
_ND = 4
_PACK = 8
_BM = 256
_BN = 256


def _qkv_kernel(idx_ref, x_ref, w_ref, qw_ref, sc_ref, b_ref, out_ref):
    xb = x_ref[...]
    wb = w_ref[...]
    acc = lax.dot_general(xb, wb, (((1,), (1,)), ((), ())),
                          preferred_element_type=jnp.float32)
    idx = idx_ref[...]
    rs8 = 8.0 * jnp.sum(xb, axis=1, keepdims=True)
    for d in range(_ND):
        qd = qw_ref[d]
        wd = jnp.concatenate(
            [((qd >> (4 * n)) & 15) for n in range(_PACK)], axis=1
        ).astype(jnp.float32)
        m = idx == d
        xm = jnp.where(m, xb, 0.0)
        part = lax.dot_general(xm, wd, (((1,), (1,)), ((), ())),
                               preferred_element_type=jnp.float32)
        corr = jnp.where(m, rs8, 0.0)
        acc = acc + (part - corr) * sc_ref[0, d:d + 1, :]
    out_ref[...] = acc + b_ref[0]


@jax.jit
def kernel(x, indices, W, b, qw_q, qw_k, qw_v, sc_q, sc_k, sc_v):
    xp = x.reshape(_T, _D // _PACK, _PACK).transpose(0, 2, 1).reshape(_T, _D)
    Wp = W.reshape(_OUT, _D // _PACK, _PACK).transpose(0, 2, 1).reshape(_OUT, _D)
    qw = jnp.concatenate([qw_q, qw_k, qw_v], axis=1)
    sc = jnp.concatenate([sc_q, sc_k, sc_v], axis=1)[..., 0]
    scr = sc.reshape(_ND, _OUT // _BN, _BN).transpose(1, 0, 2)
    idx2 = indices.reshape(_T, 1)
    b3 = b.reshape(_OUT // _BN, 1, _BN)

    grid = (_T // _BM, _OUT // _BN)
    out = pl.pallas_call(
        _qkv_kernel,
        grid=grid,
        in_specs=[
            pl.BlockSpec((_BM, 1), lambda i, j: (i, 0)),
            pl.BlockSpec((_BM, _D), lambda i, j: (i, 0)),
            pl.BlockSpec((_BN, _D), lambda i, j: (j, 0)),
            pl.BlockSpec((_ND, _BN, _D // _PACK), lambda i, j: (0, j, 0)),
            pl.BlockSpec((1, _ND, _BN), lambda i, j: (j, 0, 0)),
            pl.BlockSpec((1, 1, _BN), lambda i, j: (j, 0, 0)),
        ],
        out_specs=pl.BlockSpec((_BM, _BN), lambda i, j: (i, j)),
        out_shape=jax.ShapeDtypeStruct((_T, _OUT), jnp.float32),
    )(idx2, xp, Wp, qw, scr, b3)
    return out
